# baseline (speedup 1.0000x reference)
import functools

import jax
import jax.numpy as jnp
import numpy as np
from jax import lax
from jax.experimental import pallas as pl
from jax.experimental.pallas import tpu as pltpu
from jax.experimental.pallas import tpu_sc as plsc
from jax.sharding import PartitionSpec as P

B, N, C_IN, M, K, OUT = 8, 8192, 64, 2048, 32, 128
D_CAT = 3 + C_IN
DPAD = 80
EPS = 1e-5
MT = 512
MT2 = 256


def _knn_body(centers_ref, xyzt_ref, cent_out_ref, idx_ref):
    b = pl.program_id(0)
    c = centers_ref[0]
    xt = xyzt_ref[0]
    cent_out_ref[0] = c
    cn2 = c[:, 0:1] ** 2 + c[:, 1:2] ** 2 + c[:, 2:3] ** 2
    xn2 = xt[0:1, :] ** 2 + xt[1:2, :] ** 2 + xt[2:3, :] ** 2
    dot = jnp.dot(c, xt, preferred_element_type=jnp.float32)
    d2 = (cn2 + xn2) - 2.0 * dot
    iota = lax.broadcasted_iota(jnp.int32, (MT, N), 1)
    work = d2
    cols = []
    for _ in range(K):
        rowmin = jnp.min(work, axis=1, keepdims=True)
        cand = jnp.where(work <= rowmin, iota, N)
        sel = jnp.min(cand, axis=1, keepdims=True)
        cols.append(sel)
        work = jnp.where(cand == sel, jnp.inf, work)
    idx_ref[0] = jnp.concatenate(cols, axis=1) + b * N


def _knn(centers, xyzt, bl):
    return pl.pallas_call(
        _knn_body,
        grid=(bl, M // MT),
        in_specs=[
            pl.BlockSpec((1, MT, 3), lambda b, i: (b, i, 0)),
            pl.BlockSpec((1, 3, N), lambda b, i: (b, 0, 0)),
        ],
        out_specs=[
            pl.BlockSpec((1, MT, 3), lambda b, i: (b, i, 0)),
            pl.BlockSpec((1, MT, K), lambda b, i: (b, i, 0)),
        ],
        out_shape=[
            jax.ShapeDtypeStruct((bl, M, 3), jnp.float32),
            jax.ShapeDtypeStruct((bl, M, K), jnp.int32),
        ],
        compiler_params=pltpu.CompilerParams(
            dimension_semantics=("parallel", "parallel")),
    )(centers, xyzt)


_NC, _NS = 2, 16
_NW = _NC * _NS
_CH = 512


@functools.lru_cache(maxsize=2)
def _gather_kernel(bl):
    mesh = plsc.VectorSubcoreMesh(core_axis_name="c", subcore_axis_name="s")
    bmk = bl * M * K
    per_w = bmk // _NW
    nchunk = per_w // _CH

    @functools.partial(
        pl.kernel,
        mesh=mesh,
        out_type=jax.ShapeDtypeStruct((bmk, DPAD), jnp.float32),
        scratch_types=[
            pltpu.VMEM((2, _CH), jnp.int32),
            pltpu.VMEM((2, _CH, DPAD), jnp.float32),
            pltpu.SemaphoreType.DMA,
            pltpu.SemaphoreType.DMA,
        ],
        compiler_params=pltpu.CompilerParams(use_tc_tiling_on_sc=False),
    )
    def _gather(table_hbm, idx_hbm, out_hbm, idx_v, rows_v, sem0, sem1):
        wid = lax.axis_index("s") * _NC + lax.axis_index("c")
        sems = (sem0, sem1)

        def issue(i):
            p = i % 2
            base = wid * per_w + i * _CH
            pltpu.sync_copy(idx_hbm.at[pl.ds(base, _CH)], idx_v.at[p])
            return pltpu.async_copy(table_hbm.at[idx_v.at[p]],
                                    rows_v.at[p], sems[p])

        handles = {0: issue(0)}
        for i in range(nchunk):
            if i + 1 < nchunk:
                handles[i + 1] = issue(i + 1)
            handles[i].wait()
            base = wid * per_w + i * _CH
            pltpu.sync_copy(rows_v.at[i % 2], out_hbm.at[pl.ds(base, _CH)])

    return _gather


def _mlp_body(g_ref, cp_ref, wf_ref, wacat_ref, waf_ref, pvec_ref, out_ref):
    g = g_ref[0]
    cp = cp_ref[0]
    x = (g - cp[:, None, :]).reshape(MT2 * K, DPAD)
    bf = pvec_ref[0, :]
    betaf = pvec_ref[1, :]
    ba = pvec_ref[2, :]
    betaa = pvec_ref[3, :]
    sf = pvec_ref[4, :]
    sa = pvec_ref[5, :]

    fl = jnp.dot(x, wf_ref[...], preferred_element_type=jnp.float32) + bf
    fp = jnp.maximum(fl * sf + betaf, 0.0)
    fp3 = fp.reshape(MT2, K, OUT)
    fm = jnp.mean(fp3, axis=1, keepdims=True)
    fc = (fp3 - fm).reshape(MT2 * K, OUT)
    al = (jnp.dot(x, wacat_ref[...], preferred_element_type=jnp.float32)
          + jnp.dot(fc, waf_ref[...], preferred_element_type=jnp.float32) + ba)
    aa = jnp.maximum(al * sa + betaa, 0.0)
    a3 = aa.reshape(MT2, K, OUT)
    amax = jnp.max(a3, axis=1, keepdims=True)
    e = jnp.exp(a3 - amax)
    s = jnp.sum(e, axis=1, keepdims=True)
    w = e / s
    out_ref[0] = jnp.sum(w * fp3, axis=1)


def _mlp(g4, cpad, wf, wacat, waf, pvec, bl):
    return pl.pallas_call(
        _mlp_body,
        grid=(bl, M // MT2),
        in_specs=[
            pl.BlockSpec((1, MT2, K, DPAD), lambda b, i: (b, i, 0, 0)),
            pl.BlockSpec((1, MT2, DPAD), lambda b, i: (b, i, 0)),
            pl.BlockSpec((DPAD, OUT), lambda b, i: (0, 0)),
            pl.BlockSpec((DPAD, OUT), lambda b, i: (0, 0)),
            pl.BlockSpec((OUT, OUT), lambda b, i: (0, 0)),
            pl.BlockSpec((8, OUT), lambda b, i: (0, 0)),
        ],
        out_specs=pl.BlockSpec((1, MT2, OUT), lambda b, i: (b, i, 0)),
        out_shape=jax.ShapeDtypeStruct((bl, M, OUT), jnp.float32),
        compiler_params=pltpu.CompilerParams(
            dimension_semantics=("parallel", "parallel")),
    )(g4, cpad, wf, wacat, waf, pvec)


def _half(xyz, feat_in, wf_p, wacat, waf, pvec):
    bl = xyz.shape[0]
    idx_center = jnp.linspace(0.0, N - 1, M).astype(jnp.int32)
    centers = xyz[:, idx_center, :]
    xyzt = jnp.transpose(xyz, (0, 2, 1))

    centers_out, idx = _knn(centers, xyzt, bl)

    feat_t = jnp.transpose(feat_in, (0, 2, 1))
    table = jnp.concatenate(
        [xyz, feat_t, jnp.zeros((bl, N, DPAD - D_CAT), jnp.float32)], axis=-1)
    table2 = table.reshape(bl * N, DPAD)
    g_flat = _gather_kernel(bl)(table2, idx.reshape(bl * M * K))
    g4 = g_flat.reshape(bl, M, K, DPAD)

    cpad = jnp.concatenate(
        [centers_out, jnp.zeros((bl, M, DPAD - 3), jnp.float32)], axis=-1)
    f_region = _mlp(g4, cpad, wf_p, wacat, waf, pvec, bl)
    return centers_out, f_region


def _pipeline(xyz, feat_in, Wf, bf, gf, betaf, Wa, ba, ga, betaa):
    bl = xyz.shape[0]
    inv = float(1.0 / np.sqrt(1.0 + EPS))
    wf_p = jnp.zeros((DPAD, OUT), jnp.float32).at[:D_CAT, :].set(Wf.T)
    wacat = jnp.zeros((DPAD, OUT), jnp.float32).at[:D_CAT, :].set(Wa[:, :D_CAT].T)
    waf = Wa[:, D_CAT:].T
    pvec = jnp.stack([bf, betaf, ba, betaa, gf * inv, ga * inv,
                      jnp.zeros_like(bf), jnp.zeros_like(bf)], axis=0)

    nh = 2 if bl % 2 == 0 else 1
    h = bl // nh
    couts, fouts = [], []
    for s in range(nh):
        c, f = _half(xyz[s * h:(s + 1) * h], feat_in[s * h:(s + 1) * h],
                     wf_p, wacat, waf, pvec)
        couts.append(c)
        fouts.append(f)
    centers_out = jnp.concatenate(couts, axis=0) if nh > 1 else couts[0]
    f_region = jnp.concatenate(fouts, axis=0) if nh > 1 else fouts[0]
    return centers_out, jnp.transpose(f_region, (0, 2, 1))


def kernel(xyz, feat_in, Wf, bf, gf, betaf, Wa, ba, ga, betaa):
    devs = jax.devices()
    ndev = 2 if (len(devs) >= 2 and B % 2 == 0) else 1
    if ndev == 1:
        return _pipeline(xyz, feat_in, Wf, bf, gf, betaf, Wa, ba, ga, betaa)
    mesh = jax.sharding.Mesh(np.array(devs[:2]), ("d",))
    pd = P("d")
    pr = P()
    return jax.shard_map(
        _pipeline,
        mesh=mesh,
        in_specs=(pd, pd, pr, pr, pr, pr, pr, pr, pr, pr),
        out_specs=(pd, pd),
        check_vma=False,
    )(xyz, feat_in, Wf, bf, gf, betaf, Wa, ba, ga, betaa)


# speedup vs baseline: 1.0099x; 1.0099x over previous
import functools

import jax
import jax.numpy as jnp
import numpy as np
from jax import lax
from jax.experimental import pallas as pl
from jax.experimental.pallas import tpu as pltpu
from jax.experimental.pallas import tpu_sc as plsc
from jax.sharding import PartitionSpec as P

B, N, C_IN, M, K, OUT = 8, 8192, 64, 2048, 32, 128
D_CAT = 3 + C_IN
DPAD = 80
EPS = 1e-5
MT = 512
MT2 = 256


def _knn_body(centers_ref, xyzt_ref, cent_out_ref, idx_ref):
    b = pl.program_id(0)
    c = centers_ref[0]
    xt = xyzt_ref[0]
    cent_out_ref[0] = c
    cn2 = c[:, 0:1] ** 2 + c[:, 1:2] ** 2 + c[:, 2:3] ** 2
    xn2 = xt[0:1, :] ** 2 + xt[1:2, :] ** 2 + xt[2:3, :] ** 2
    dot = jnp.dot(c, xt, preferred_element_type=jnp.float32)
    d2 = (cn2 + xn2) - 2.0 * dot
    iota = lax.broadcasted_iota(jnp.int32, (MT, N), 1)
    work = d2
    cols = []
    for _ in range(K):
        rowmin = jnp.min(work, axis=1, keepdims=True)
        cand = jnp.where(work <= rowmin, iota, N)
        sel = jnp.min(cand, axis=1, keepdims=True)
        cols.append(sel)
        work = jnp.where(cand == sel, jnp.inf, work)
    idx_ref[0] = jnp.concatenate(cols, axis=1) + b * N


def _knn(centers, xyzt, bl):
    return pl.pallas_call(
        _knn_body,
        grid=(bl, M // MT),
        in_specs=[
            pl.BlockSpec((1, MT, 3), lambda b, i: (b, i, 0)),
            pl.BlockSpec((1, 3, N), lambda b, i: (b, 0, 0)),
        ],
        out_specs=[
            pl.BlockSpec((1, MT, 3), lambda b, i: (b, i, 0)),
            pl.BlockSpec((1, MT, K), lambda b, i: (b, i, 0)),
        ],
        out_shape=[
            jax.ShapeDtypeStruct((bl, M, 3), jnp.float32),
            jax.ShapeDtypeStruct((bl, M, K), jnp.int32),
        ],
        compiler_params=pltpu.CompilerParams(
            dimension_semantics=("parallel", "parallel")),
    )(centers, xyzt)


_NC, _NS = 2, 16
_NW = _NC * _NS
_CH = 512


@functools.lru_cache(maxsize=2)
def _gather_kernel(bl):
    mesh = plsc.VectorSubcoreMesh(core_axis_name="c", subcore_axis_name="s")
    bmk = bl * M * K
    per_w = bmk // _NW
    nchunk = per_w // _CH

    @functools.partial(
        pl.kernel,
        mesh=mesh,
        out_type=jax.ShapeDtypeStruct((bmk, DPAD), jnp.float32),
        scratch_types=[
            pltpu.VMEM((2, _CH), jnp.int32),
            pltpu.VMEM((2, _CH, DPAD), jnp.float32),
            pltpu.SemaphoreType.DMA,
            pltpu.SemaphoreType.DMA,
        ],
        compiler_params=pltpu.CompilerParams(use_tc_tiling_on_sc=False),
    )
    def _gather(table_hbm, idx_hbm, out_hbm, idx_v, rows_v, sem0, sem1):
        wid = lax.axis_index("s") * _NC + lax.axis_index("c")
        sems = (sem0, sem1)

        def issue(i):
            p = i % 2
            base = wid * per_w + i * _CH
            pltpu.sync_copy(idx_hbm.at[pl.ds(base, _CH)], idx_v.at[p])
            return pltpu.async_copy(table_hbm.at[idx_v.at[p]],
                                    rows_v.at[p], sems[p])

        handles = {0: issue(0)}
        for i in range(nchunk):
            if i + 1 < nchunk:
                handles[i + 1] = issue(i + 1)
            handles[i].wait()
            base = wid * per_w + i * _CH
            pltpu.sync_copy(rows_v.at[i % 2], out_hbm.at[pl.ds(base, _CH)])

    return _gather


def _mlp_body(g_ref, cp_ref, wf_ref, wacat_ref, waf_ref, pvec_ref, out_ref):
    g = g_ref[0]
    cp = cp_ref[0]
    x = (g - cp[:, None, :]).reshape(MT2 * K, DPAD)
    bf = pvec_ref[0, :]
    betaf = pvec_ref[1, :]
    ba = pvec_ref[2, :]
    betaa = pvec_ref[3, :]
    sf = pvec_ref[4, :]
    sa = pvec_ref[5, :]

    fl = jnp.dot(x, wf_ref[...], preferred_element_type=jnp.float32) + bf
    fp = jnp.maximum(fl * sf + betaf, 0.0)
    fp3 = fp.reshape(MT2, K, OUT)
    fm = jnp.mean(fp3, axis=1, keepdims=True)
    fc = (fp3 - fm).reshape(MT2 * K, OUT)
    al = (jnp.dot(x, wacat_ref[...], preferred_element_type=jnp.float32)
          + jnp.dot(fc, waf_ref[...], preferred_element_type=jnp.float32) + ba)
    aa = jnp.maximum(al * sa + betaa, 0.0)
    a3 = aa.reshape(MT2, K, OUT)
    amax = jnp.max(a3, axis=1, keepdims=True)
    e = jnp.exp(a3 - amax)
    s = jnp.sum(e, axis=1, keepdims=True)
    w = e / s
    out_ref[0] = jnp.sum(w * fp3, axis=1)


def _mlp(g4, cpad, wf, wacat, waf, pvec, bl):
    return pl.pallas_call(
        _mlp_body,
        grid=(bl, M // MT2),
        in_specs=[
            pl.BlockSpec((1, MT2, K, DPAD), lambda b, i: (b, i, 0, 0)),
            pl.BlockSpec((1, MT2, DPAD), lambda b, i: (b, i, 0)),
            pl.BlockSpec((DPAD, OUT), lambda b, i: (0, 0)),
            pl.BlockSpec((DPAD, OUT), lambda b, i: (0, 0)),
            pl.BlockSpec((OUT, OUT), lambda b, i: (0, 0)),
            pl.BlockSpec((8, OUT), lambda b, i: (0, 0)),
        ],
        out_specs=pl.BlockSpec((1, MT2, OUT), lambda b, i: (b, i, 0)),
        out_shape=jax.ShapeDtypeStruct((bl, M, OUT), jnp.float32),
        compiler_params=pltpu.CompilerParams(
            dimension_semantics=("parallel", "parallel")),
    )(g4, cpad, wf, wacat, waf, pvec)


def _half(xyz, feat_in, wf_p, wacat, waf, pvec):
    bl = xyz.shape[0]
    idx_center = jnp.linspace(0.0, N - 1, M).astype(jnp.int32)
    centers = xyz[:, idx_center, :]
    xyzt = jnp.transpose(xyz, (0, 2, 1))

    centers_out, idx = _knn(centers, xyzt, bl)

    feat_t = jnp.transpose(feat_in, (0, 2, 1))
    table = jnp.concatenate(
        [xyz, feat_t, jnp.zeros((bl, N, DPAD - D_CAT), jnp.float32)], axis=-1)
    table2 = table.reshape(bl * N, DPAD)
    g_flat = _gather_kernel(bl)(table2, idx.reshape(bl * M * K))
    g4 = g_flat.reshape(bl, M, K, DPAD)

    cpad = jnp.concatenate(
        [centers_out, jnp.zeros((bl, M, DPAD - 3), jnp.float32)], axis=-1)
    f_region = _mlp(g4, cpad, wf_p, wacat, waf, pvec, bl)
    return centers_out, f_region


def _pipeline(xyz, feat_in, Wf, bf, gf, betaf, Wa, ba, ga, betaa):
    bl = xyz.shape[0]
    inv = float(1.0 / np.sqrt(1.0 + EPS))
    wf_p = jnp.zeros((DPAD, OUT), jnp.float32).at[:D_CAT, :].set(Wf.T)
    wacat = jnp.zeros((DPAD, OUT), jnp.float32).at[:D_CAT, :].set(Wa[:, :D_CAT].T)
    waf = Wa[:, D_CAT:].T
    pvec = jnp.stack([bf, betaf, ba, betaa, gf * inv, ga * inv,
                      jnp.zeros_like(bf), jnp.zeros_like(bf)], axis=0)

    centers_out, f_region = _half(xyz, feat_in, wf_p, wacat, waf, pvec)
    return centers_out, jnp.transpose(f_region, (0, 2, 1))


def kernel(xyz, feat_in, Wf, bf, gf, betaf, Wa, ba, ga, betaa):
    devs = jax.devices()
    ndev = 2 if (len(devs) >= 2 and B % 2 == 0) else 1
    if ndev == 1:
        return _pipeline(xyz, feat_in, Wf, bf, gf, betaf, Wa, ba, ga, betaa)
    mesh = jax.sharding.Mesh(np.array(devs[:2]), ("d",))
    pd = P("d")
    pr = P()
    return jax.shard_map(
        _pipeline,
        mesh=mesh,
        in_specs=(pd, pd, pr, pr, pr, pr, pr, pr, pr, pr),
        out_specs=(pd, pd),
        check_vma=False,
    )(xyz, feat_in, Wf, bf, gf, betaf, Wa, ba, ga, betaa)
